# Initial kernel scaffold; baseline (speedup 1.0000x reference)
import jax
import jax.numpy as jnp
from jax.experimental import pallas as pl


def kernel(x, edge_index, edge_attr, u, action_mask, W1, b1, Wl, bl, W2, b2, W3, b3, W4, b4):
    raise NotImplementedError("write your pallas kernel here")


# speedup vs baseline: 203.0702x; 203.0702x over previous
import functools

import jax
import jax.numpy as jnp
from jax import lax
from jax.experimental import pallas as pl
from jax.experimental.pallas import tpu as pltpu
from jax.experimental.pallas import tpu_sc as plsc

N = 10000
E = 160000
D = 256
H = 22
O = 11

NC, NS, L = 2, 16, 16
NW = NC * NS
EPT = 5008
EP = EPT * NW

N_SLICES = 4
N_GROUPS = 8
SLICE_E = E // N_SLICES
CHUNK = 2000
N_CHUNKS = SLICE_E // CHUNK

_mesh = plsc.VectorSubcoreMesh(
    core_axis_name="c", subcore_axis_name="s", num_cores=NC, num_subcores=NS)
_sc_params = pltpu.CompilerParams(needs_layout_passes=False)


_I = jnp.int32


def _wid():
    return lax.axis_index("s") * _I(NC) + lax.axis_index("c")


@functools.partial(
    pl.kernel,
    out_type=jax.ShapeDtypeStruct((NW * N,), jnp.float32),
    compiler_params=_sc_params,
    mesh=_mesh,
    scratch_types=[
        pltpu.VMEM((N,), jnp.float32),
        pltpu.VMEM((EPT,), jnp.int32),
        pltpu.VMEM((EPT,), jnp.float32),
    ],
)
def _sc_deg(c_hbm, w_hbm, out_hbm, deg_v, c_v, w_v):
    wid = _wid()
    base = wid * _I(EPT)
    pltpu.sync_copy(c_hbm.at[pl.ds(base, EPT)], c_v)
    pltpu.sync_copy(w_hbm.at[pl.ds(base, EPT)], w_v)

    def zbody(i, carry):
        deg_v[pl.ds(i * _I(L), L)] = jnp.zeros((L,), jnp.float32)
        return carry

    lax.fori_loop(_I(0), _I(N // L), zbody, _I(0))

    def body(i, carry):
        c16 = c_v[pl.ds(i * _I(L), L)]
        w16 = w_v[pl.ds(i * _I(L), L)]
        plsc.addupdate_scatter(deg_v, [c16], w16)
        return carry

    lax.fori_loop(_I(0), _I(EPT // L), body, _I(0))
    pltpu.sync_copy(deg_v, out_hbm.at[pl.ds(wid * _I(N), N)])


@functools.partial(
    pl.kernel,
    out_type=jax.ShapeDtypeStruct((EP,), jnp.float32),
    compiler_params=_sc_params,
    mesh=_mesh,
    scratch_types=[
        pltpu.VMEM((N,), jnp.float32),
        pltpu.VMEM((EPT,), jnp.int32),
        pltpu.VMEM((EPT,), jnp.int32),
        pltpu.VMEM((EPT,), jnp.float32),
        pltpu.VMEM((EPT,), jnp.float32),
    ],
)
def _sc_norm(r_hbm, c_hbm, w_hbm, dis_hbm, norm_hbm, dis_v, r_v, c_v, w_v, n_v):
    wid = _wid()
    base = wid * _I(EPT)
    pltpu.sync_copy(dis_hbm, dis_v)
    pltpu.sync_copy(r_hbm.at[pl.ds(base, EPT)], r_v)
    pltpu.sync_copy(c_hbm.at[pl.ds(base, EPT)], c_v)
    pltpu.sync_copy(w_hbm.at[pl.ds(base, EPT)], w_v)

    def body(i, carry):
        sl = pl.ds(i * _I(L), L)
        r16 = r_v[sl]
        c16 = c_v[sl]
        w16 = w_v[sl]
        dr = plsc.load_gather(dis_v, [r16])
        dc = plsc.load_gather(dis_v, [c16])
        n_v[sl] = dr * w16 * dc
        return carry

    lax.fori_loop(_I(0), _I(EPT // L), body, _I(0))
    pltpu.sync_copy(n_v, norm_hbm.at[pl.ds(base, EPT)])


def _make_sc_layer(ncp):
    cpg = ncp // N_GROUPS
    scratch = ([pltpu.VMEM((N,), jnp.float32)] * (2 * cpg)) + [
        pltpu.VMEM((CHUNK,), jnp.int32),
        pltpu.VMEM((CHUNK,), jnp.int32),
        pltpu.VMEM((CHUNK,), jnp.float32),
    ]

    @functools.partial(
        pl.kernel,
        out_type=jax.ShapeDtypeStruct((N_SLICES * ncp * N,), jnp.float32),
        compiler_params=_sc_params,
        mesh=_mesh,
        scratch_types=scratch,
    )
    def sc_layer(g_hbm, r_hbm, c_hbm, norm_hbm, out_hbm, *refs):
        g_cols = refs[:cpg]
        o_cols = refs[cpg:2 * cpg]
        r_v, c_v, n_v = refs[2 * cpg:2 * cpg + 3]
        wid = _wid()
        sl_id = wid % _I(N_SLICES)
        gr = wid // _I(N_SLICES)
        for k in range(cpg):
            pltpu.sync_copy(g_hbm.at[pl.ds((gr * _I(cpg) + _I(k)) * _I(N), N)], g_cols[k])

        def zbody(i, carry):
            z = jnp.zeros((L,), jnp.float32)
            for k in range(cpg):
                o_cols[k][pl.ds(i * _I(L), L)] = z
            return carry

        lax.fori_loop(_I(0), _I(N // L), zbody, _I(0))

        ebase = sl_id * _I(SLICE_E)

        def chunk_body(ch, carry):
            cb = ebase + ch * _I(CHUNK)
            pltpu.sync_copy(r_hbm.at[pl.ds(cb, CHUNK)], r_v)
            pltpu.sync_copy(c_hbm.at[pl.ds(cb, CHUNK)], c_v)
            pltpu.sync_copy(norm_hbm.at[pl.ds(cb, CHUNK)], n_v)

            def ibody(i, icarry):
                s = pl.ds(i * _I(L), L)
                r16 = r_v[s]
                c16 = c_v[s]
                n16 = n_v[s]
                for k in range(cpg):
                    v = plsc.load_gather(g_cols[k], [r16]) * n16
                    plsc.addupdate_scatter(o_cols[k], [c16], v)
                return icarry

            lax.fori_loop(_I(0), _I(CHUNK // L), ibody, _I(0))
            return carry

        lax.fori_loop(_I(0), _I(N_CHUNKS), chunk_body, _I(0))
        for k in range(cpg):
            off = (sl_id * _I(ncp) + gr * _I(cpg) + _I(k)) * _I(N)
            pltpu.sync_copy(o_cols[k], out_hbm.at[pl.ds(off, N)])

    return sc_layer


_sc_layer24 = _make_sc_layer(24)
_sc_layer16 = _make_sc_layer(16)


def _tc_prep(degp, xT, w1tp):
    def body(degp_ref, xT_ref, w_ref, g_ref, dis_ref):
        deg = jnp.sum(degp_ref[...], axis=0, keepdims=True) + 1.0
        dis_ref[...] = lax.rsqrt(deg)
        g_ref[...] = jnp.dot(w_ref[...], xT_ref[...],
                             preferred_element_type=jnp.float32, precision=lax.Precision.HIGHEST)

    return pl.pallas_call(
        body,
        out_shape=[
            jax.ShapeDtypeStruct((24, N), jnp.float32),
            jax.ShapeDtypeStruct((1, N), jnp.float32),
        ],
    )(degp, xT, w1tp)


def _tc_mid(P, gT, dis, wnt, bcol, ncp_in, ncp_out, ucol=None, wlt=None,
            blcol=None):
    def body(*refs):
        if ucol is None:
            P_ref, g_ref, dis_ref, w_ref, b_ref, o_ref = refs
        else:
            P_ref, g_ref, dis_ref, w_ref, b_ref, u_ref, wl_ref, bl_ref, o_ref = refs
        dis_v = dis_ref[...]
        s = jnp.sum(P_ref[...], axis=0) + dis_v * dis_v * g_ref[...] + b_ref[...]
        h = jnp.maximum(s, 0.0)
        if ucol is not None:
            ut = jnp.dot(wl_ref[...], u_ref[...],
                         preferred_element_type=jnp.float32, precision=lax.Precision.HIGHEST) + bl_ref[...]
            h = h + jnp.maximum(ut, 0.0)
        o_ref[...] = jnp.dot(w_ref[...], h, preferred_element_type=jnp.float32, precision=lax.Precision.HIGHEST)

    args = [P.reshape(N_SLICES, ncp_in, N), gT, dis, wnt, bcol]
    if ucol is not None:
        args += [ucol, wlt, blcol]
    return pl.pallas_call(
        body,
        out_shape=jax.ShapeDtypeStruct((ncp_out, N), jnp.float32),
    )(*args)


def _tc_final(P, gT, dis, bcol, maskT):
    def body(P_ref, g_ref, dis_ref, b_ref, m_ref, o_ref):
        dis_v = dis_ref[...]
        s = jnp.sum(P_ref[...], axis=0) + dis_v * dis_v * g_ref[...] + b_ref[...]
        o_ref[...] = s[:O] + (m_ref[...] - 1.0) * 1000.0

    return pl.pallas_call(
        body,
        out_shape=jax.ShapeDtypeStruct((O, N), jnp.float32),
    )(P.reshape(N_SLICES, 16, N), gT, dis, bcol, maskT)


def _padT(W, rows, cols):
    out = jnp.zeros((rows, cols), jnp.float32)
    return out.at[:W.shape[1], :W.shape[0]].set(W.T.astype(jnp.float32))


def _col(b, rows):
    out = jnp.zeros((rows, 1), jnp.float32)
    return out.at[:b.shape[0], 0].set(b.astype(jnp.float32))


def kernel(x, edge_index, edge_attr, u, action_mask,
           W1, b1, Wl, bl, W2, b2, W3, b3, W4, b4):
    f32 = jnp.float32
    i32 = jnp.int32
    pad = EP - E
    r = jnp.concatenate([edge_index[0].astype(i32), jnp.zeros((pad,), i32)])
    c = jnp.concatenate([edge_index[1].astype(i32), jnp.zeros((pad,), i32)])
    w = jnp.concatenate([edge_attr.astype(f32), jnp.zeros((pad,), f32)])

    xT = x.astype(f32).T
    w1tp = _padT(W1, 24, D)
    w2tp = _padT(W2, 24, 24)
    w3tp = _padT(W3, 24, 24)
    w4tp = _padT(W4, 16, 24)
    wltp = _padT(Wl, 24, 24)
    b1c, b2c, b3c = _col(b1, 24), _col(b2, 24), _col(b3, 24)
    b4c = _col(b4, 16)
    blc = _col(bl, 24)
    uc = _col(u, 24)
    maskT = action_mask.astype(f32).T

    degp = _sc_deg(c, w).reshape(NW, N)
    g1T, dis = _tc_prep(degp, xT, w1tp)
    norm = _sc_norm(r, c, w, dis.reshape(N))

    P1 = _sc_layer24(g1T.reshape(-1), r, c, norm)
    g2T = _tc_mid(P1, g1T, dis, w2tp, b1c, 24, 24, ucol=uc, wlt=wltp, blcol=blc)
    P2 = _sc_layer24(g2T.reshape(-1), r, c, norm)
    g3T = _tc_mid(P2, g2T, dis, w3tp, b2c, 24, 24)
    P3 = _sc_layer24(g3T.reshape(-1), r, c, norm)
    g4T = _tc_mid(P3, g3T, dis, w4tp, b3c, 24, 16)
    P4 = _sc_layer16(g4T.reshape(-1), r, c, norm)
    outT = _tc_final(P4, g4T, dis, b4c, maskT)
    return outT.T.astype(jnp.float64)


# speedup vs baseline: 298.7891x; 1.4714x over previous
import functools

import jax
import jax.numpy as jnp
from jax import lax
from jax.experimental import pallas as pl
from jax.experimental.pallas import tpu as pltpu
from jax.experimental.pallas import tpu_sc as plsc

N = 10000
E = 160000
D = 256
H = 22
O = 11

NC, NS, L = 2, 16, 16
NW = NC * NS
EPT = 5008
EP = EPT * NW

N_SLICES = 4
N_GROUPS = 8
SLICE_E = E // N_SLICES
CHUNK = 4000
N_CHUNKS = SLICE_E // CHUNK

_mesh = plsc.VectorSubcoreMesh(
    core_axis_name="c", subcore_axis_name="s", num_cores=NC, num_subcores=NS)
_sc_params = pltpu.CompilerParams(needs_layout_passes=False)


_I = jnp.int32


def _wid():
    return lax.axis_index("s") * _I(NC) + lax.axis_index("c")


@functools.partial(
    pl.kernel,
    out_type=jax.ShapeDtypeStruct((NW * N,), jnp.float32),
    compiler_params=_sc_params,
    mesh=_mesh,
    scratch_types=[
        pltpu.VMEM((N,), jnp.float32),
        pltpu.VMEM((EPT,), jnp.int32),
        pltpu.VMEM((EPT,), jnp.float32),
    ],
)
def _sc_deg(c_hbm, w_hbm, out_hbm, deg_v, c_v, w_v):
    wid = _wid()
    base = wid * _I(EPT)
    pltpu.sync_copy(c_hbm.at[pl.ds(base, EPT)], c_v)
    pltpu.sync_copy(w_hbm.at[pl.ds(base, EPT)], w_v)

    def zbody(i, carry):
        deg_v[pl.ds(i * _I(L), L)] = jnp.zeros((L,), jnp.float32)
        return carry

    lax.fori_loop(_I(0), _I(N // L), zbody, _I(0))

    def body(i, carry):
        c16 = c_v[pl.ds(i * _I(L), L)]
        w16 = w_v[pl.ds(i * _I(L), L)]
        plsc.addupdate_scatter(deg_v, [c16], w16)
        return carry

    lax.fori_loop(_I(0), _I(EPT // L), body, _I(0))
    pltpu.sync_copy(deg_v, out_hbm.at[pl.ds(wid * _I(N), N)])


@functools.partial(
    pl.kernel,
    out_type=jax.ShapeDtypeStruct((EP,), jnp.float32),
    compiler_params=_sc_params,
    mesh=_mesh,
    scratch_types=[
        pltpu.VMEM((N,), jnp.float32),
        pltpu.VMEM((EPT,), jnp.int32),
        pltpu.VMEM((EPT,), jnp.int32),
        pltpu.VMEM((EPT,), jnp.float32),
        pltpu.VMEM((EPT,), jnp.float32),
    ],
)
def _sc_norm(r_hbm, c_hbm, w_hbm, dis_hbm, norm_hbm, dis_v, r_v, c_v, w_v, n_v):
    wid = _wid()
    base = wid * _I(EPT)
    pltpu.sync_copy(dis_hbm, dis_v)
    pltpu.sync_copy(r_hbm.at[pl.ds(base, EPT)], r_v)
    pltpu.sync_copy(c_hbm.at[pl.ds(base, EPT)], c_v)
    pltpu.sync_copy(w_hbm.at[pl.ds(base, EPT)], w_v)

    def body(i, carry):
        sl = pl.ds(i * _I(L), L)
        r16 = r_v[sl]
        c16 = c_v[sl]
        w16 = w_v[sl]
        dr = plsc.load_gather(dis_v, [r16])
        dc = plsc.load_gather(dis_v, [c16])
        n_v[sl] = dr * w16 * dc
        return carry

    lax.fori_loop(_I(0), _I(EPT // L), body, _I(0))
    pltpu.sync_copy(n_v, norm_hbm.at[pl.ds(base, EPT)])


def _make_sc_layer(ncp):
    cpg = ncp // N_GROUPS
    scratch = ([pltpu.VMEM((N,), jnp.float32)] * (2 * cpg)) + (
        [pltpu.VMEM((CHUNK,), jnp.int32)] * 4) + (
        [pltpu.VMEM((CHUNK,), jnp.float32)] * 2) + [
        pltpu.SemaphoreType.DMA,
        pltpu.SemaphoreType.DMA,
        pltpu.SemaphoreType.DMA,
    ]

    @functools.partial(
        pl.kernel,
        out_type=jax.ShapeDtypeStruct((N_SLICES * ncp * N,), jnp.float32),
        compiler_params=_sc_params,
        mesh=_mesh,
        scratch_types=scratch,
    )
    def sc_layer(g_hbm, r_hbm, c_hbm, norm_hbm, out_hbm, *refs):
        g_cols = refs[:cpg]
        o_cols = refs[cpg:2 * cpg]
        r0, r1, c0, c1 = refs[2 * cpg:2 * cpg + 4]
        n0, n1 = refs[2 * cpg + 4:2 * cpg + 6]
        sem0, sem1, gsem = refs[2 * cpg + 6:2 * cpg + 9]
        rbuf, cbuf, nbuf, sems = (r0, r1), (c0, c1), (n0, n1), (sem0, sem1)
        wid = _wid()
        sl_id = wid % _I(N_SLICES)
        gr = wid // _I(N_SLICES)
        ebase = sl_id * _I(SLICE_E)

        def chunk_refs(ch, b):
            cb = ebase + ch * _I(CHUNK)
            return ((r_hbm.at[pl.ds(cb, CHUNK)], rbuf[b]),
                    (c_hbm.at[pl.ds(cb, CHUNK)], cbuf[b]),
                    (norm_hbm.at[pl.ds(cb, CHUNK)], nbuf[b]))

        def start(ch, b):
            for s, d in chunk_refs(ch, b):
                pltpu.async_copy(s, d, sems[b])

        def wait(ch, b):
            for s, d in chunk_refs(ch, b):
                pltpu.make_async_copy(s, d, sems[b]).wait()

        gdescs = []
        for k in range(cpg):
            gdescs.append(pltpu.async_copy(
                g_hbm.at[pl.ds((gr * _I(cpg) + _I(k)) * _I(N), N)],
                g_cols[k], gsem))
        start(_I(0), 0)

        def zbody(i, carry):
            z = jnp.zeros((L,), jnp.float32)
            for k in range(cpg):
                o_cols[k][pl.ds(i * _I(L), L)] = z
            return carry

        lax.fori_loop(_I(0), _I(N // L), zbody, _I(0))
        for d in gdescs:
            d.wait()

        def process(b):
            def ibody(i, icarry):
                s = pl.ds(i * _I(L), L)
                r16 = rbuf[b][s]
                c16 = cbuf[b][s]
                n16 = nbuf[b][s]
                for k in range(cpg):
                    v = plsc.load_gather(g_cols[k], [r16]) * n16
                    plsc.addupdate_scatter(o_cols[k], [c16], v)
                return icarry

            lax.fori_loop(_I(0), _I(CHUNK // L), ibody, _I(0))

        def outer(t, carry):
            for b in (0, 1):
                ch = t * _I(2) + _I(b)
                nxt = ch + _I(1)

                @pl.when(nxt < _I(N_CHUNKS))
                def _():
                    start(nxt, 1 - b)

                wait(ch, b)
                process(b)
            return carry

        lax.fori_loop(_I(0), _I(N_CHUNKS // 2), outer, _I(0))
        for k in range(cpg):
            off = (sl_id * _I(ncp) + gr * _I(cpg) + _I(k)) * _I(N)
            pltpu.sync_copy(o_cols[k], out_hbm.at[pl.ds(off, N)])

    return sc_layer


_sc_layer24 = _make_sc_layer(24)
_sc_layer16 = _make_sc_layer(16)


def _tc_prep(degp, xT, w1tp):
    def body(degp_ref, xT_ref, w_ref, g_ref, dis_ref):
        deg = jnp.sum(degp_ref[...], axis=0, keepdims=True) + 1.0
        dis_ref[...] = lax.rsqrt(deg)
        g_ref[...] = jnp.dot(w_ref[...], xT_ref[...],
                             preferred_element_type=jnp.float32, precision=lax.Precision.HIGHEST)

    return pl.pallas_call(
        body,
        out_shape=[
            jax.ShapeDtypeStruct((24, N), jnp.float32),
            jax.ShapeDtypeStruct((1, N), jnp.float32),
        ],
    )(degp, xT, w1tp)


def _tc_mid(P, gT, dis, wnt, bcol, ncp_in, ncp_out, ucol=None, wlt=None,
            blcol=None):
    def body(*refs):
        if ucol is None:
            P_ref, g_ref, dis_ref, w_ref, b_ref, o_ref = refs
        else:
            P_ref, g_ref, dis_ref, w_ref, b_ref, u_ref, wl_ref, bl_ref, o_ref = refs
        dis_v = dis_ref[...]
        s = jnp.sum(P_ref[...], axis=0) + dis_v * dis_v * g_ref[...] + b_ref[...]
        h = jnp.maximum(s, 0.0)
        if ucol is not None:
            ut = jnp.dot(wl_ref[...], u_ref[...],
                         preferred_element_type=jnp.float32, precision=lax.Precision.HIGHEST) + bl_ref[...]
            h = h + jnp.maximum(ut, 0.0)
        o_ref[...] = jnp.dot(w_ref[...], h, preferred_element_type=jnp.float32, precision=lax.Precision.HIGHEST)

    args = [P.reshape(N_SLICES, ncp_in, N), gT, dis, wnt, bcol]
    if ucol is not None:
        args += [ucol, wlt, blcol]
    return pl.pallas_call(
        body,
        out_shape=jax.ShapeDtypeStruct((ncp_out, N), jnp.float32),
    )(*args)


def _tc_final(P, gT, dis, bcol, maskT):
    def body(P_ref, g_ref, dis_ref, b_ref, m_ref, o_ref):
        dis_v = dis_ref[...]
        s = jnp.sum(P_ref[...], axis=0) + dis_v * dis_v * g_ref[...] + b_ref[...]
        o_ref[...] = s[:O] + (m_ref[...] - 1.0) * 1000.0

    return pl.pallas_call(
        body,
        out_shape=jax.ShapeDtypeStruct((O, N), jnp.float32),
    )(P.reshape(N_SLICES, 16, N), gT, dis, bcol, maskT)


def _padT(W, rows, cols):
    out = jnp.zeros((rows, cols), jnp.float32)
    return out.at[:W.shape[1], :W.shape[0]].set(W.T.astype(jnp.float32))


def _col(b, rows):
    out = jnp.zeros((rows, 1), jnp.float32)
    return out.at[:b.shape[0], 0].set(b.astype(jnp.float32))


def kernel(x, edge_index, edge_attr, u, action_mask,
           W1, b1, Wl, bl, W2, b2, W3, b3, W4, b4):
    f32 = jnp.float32
    i32 = jnp.int32
    pad = EP - E
    r = jnp.concatenate([edge_index[0].astype(i32), jnp.zeros((pad,), i32)])
    c = jnp.concatenate([edge_index[1].astype(i32), jnp.zeros((pad,), i32)])
    w = jnp.concatenate([edge_attr.astype(f32), jnp.zeros((pad,), f32)])

    xT = x.astype(f32).T
    w1tp = _padT(W1, 24, D)
    w2tp = _padT(W2, 24, 24)
    w3tp = _padT(W3, 24, 24)
    w4tp = _padT(W4, 16, 24)
    wltp = _padT(Wl, 24, 24)
    b1c, b2c, b3c = _col(b1, 24), _col(b2, 24), _col(b3, 24)
    b4c = _col(b4, 16)
    blc = _col(bl, 24)
    uc = _col(u, 24)
    maskT = action_mask.astype(f32).T

    degp = _sc_deg(c, w).reshape(NW, N)
    g1T, dis = _tc_prep(degp, xT, w1tp)
    norm = _sc_norm(r, c, w, dis.reshape(N))

    P1 = _sc_layer24(g1T.reshape(-1), r, c, norm)
    g2T = _tc_mid(P1, g1T, dis, w2tp, b1c, 24, 24, ucol=uc, wlt=wltp, blcol=blc)
    P2 = _sc_layer24(g2T.reshape(-1), r, c, norm)
    g3T = _tc_mid(P2, g2T, dis, w3tp, b2c, 24, 24)
    P3 = _sc_layer24(g3T.reshape(-1), r, c, norm)
    g4T = _tc_mid(P3, g3T, dis, w4tp, b3c, 24, 16)
    P4 = _sc_layer16(g4T.reshape(-1), r, c, norm)
    outT = _tc_final(P4, g4T, dis, b4c, maskT)
    return outT.T.astype(jnp.float64)


# speedup vs baseline: 465.1578x; 1.5568x over previous
import functools

import jax
import jax.numpy as jnp
from jax import lax
from jax.experimental import pallas as pl
from jax.experimental.pallas import tpu as pltpu
from jax.experimental.pallas import tpu_sc as plsc

N = 10000
E = 160000
D = 256
H = 22
O = 11

NC, NS, L = 2, 16, 16
NW = NC * NS
EPT = 5008
EP = EPT * NW

N_SLICES = 4
N_GROUPS = 8
SLICE_E = E // N_SLICES
CHUNK = 4000
UNROLL = 5
N_CHUNKS = SLICE_E // CHUNK

_mesh = plsc.VectorSubcoreMesh(
    core_axis_name="c", subcore_axis_name="s", num_cores=NC, num_subcores=NS)
_sc_params = pltpu.CompilerParams(needs_layout_passes=False)


_I = jnp.int32


def _wid():
    return lax.axis_index("s") * _I(NC) + lax.axis_index("c")


@functools.partial(
    pl.kernel,
    out_type=jax.ShapeDtypeStruct((NW * N,), jnp.float32),
    compiler_params=_sc_params,
    mesh=_mesh,
    scratch_types=[
        pltpu.VMEM((N,), jnp.float32),
        pltpu.VMEM((EPT,), jnp.int32),
        pltpu.VMEM((EPT,), jnp.float32),
    ],
)
def _sc_deg(c_hbm, w_hbm, out_hbm, deg_v, c_v, w_v):
    wid = _wid()
    base = wid * _I(EPT)
    pltpu.sync_copy(c_hbm.at[pl.ds(base, EPT)], c_v)
    pltpu.sync_copy(w_hbm.at[pl.ds(base, EPT)], w_v)

    def zbody(i, carry):
        deg_v[pl.ds(i * _I(L), L)] = jnp.zeros((L,), jnp.float32)
        return carry

    lax.fori_loop(_I(0), _I(N // L), zbody, _I(0))

    def body(i, carry):
        c16 = c_v[pl.ds(i * _I(L), L)]
        w16 = w_v[pl.ds(i * _I(L), L)]
        plsc.addupdate_scatter(deg_v, [c16], w16)
        return carry

    lax.fori_loop(_I(0), _I(EPT // L), body, _I(0))
    pltpu.sync_copy(deg_v, out_hbm.at[pl.ds(wid * _I(N), N)])


@functools.partial(
    pl.kernel,
    out_type=jax.ShapeDtypeStruct((EP,), jnp.float32),
    compiler_params=_sc_params,
    mesh=_mesh,
    scratch_types=[
        pltpu.VMEM((N,), jnp.float32),
        pltpu.VMEM((EPT,), jnp.int32),
        pltpu.VMEM((EPT,), jnp.int32),
        pltpu.VMEM((EPT,), jnp.float32),
        pltpu.VMEM((EPT,), jnp.float32),
    ],
)
def _sc_norm(r_hbm, c_hbm, w_hbm, dis_hbm, norm_hbm, dis_v, r_v, c_v, w_v, n_v):
    wid = _wid()
    base = wid * _I(EPT)
    pltpu.sync_copy(dis_hbm, dis_v)
    pltpu.sync_copy(r_hbm.at[pl.ds(base, EPT)], r_v)
    pltpu.sync_copy(c_hbm.at[pl.ds(base, EPT)], c_v)
    pltpu.sync_copy(w_hbm.at[pl.ds(base, EPT)], w_v)

    def body(i, carry):
        sl = pl.ds(i * _I(L), L)
        r16 = r_v[sl]
        c16 = c_v[sl]
        w16 = w_v[sl]
        dr = plsc.load_gather(dis_v, [r16])
        dc = plsc.load_gather(dis_v, [c16])
        n_v[sl] = dr * w16 * dc
        return carry

    lax.fori_loop(_I(0), _I(EPT // L), body, _I(0))
    pltpu.sync_copy(n_v, norm_hbm.at[pl.ds(base, EPT)])


def _make_sc_layer(ncp):
    cpg = ncp // N_GROUPS
    scratch = ([pltpu.VMEM((N,), jnp.float32)] * (2 * cpg)) + (
        [pltpu.VMEM((CHUNK,), jnp.int32)] * 4) + (
        [pltpu.VMEM((CHUNK,), jnp.float32)] * 2) + [
        pltpu.SemaphoreType.DMA,
        pltpu.SemaphoreType.DMA,
        pltpu.SemaphoreType.DMA,
    ]

    @functools.partial(
        pl.kernel,
        out_type=jax.ShapeDtypeStruct((N_SLICES * ncp * N,), jnp.float32),
        compiler_params=_sc_params,
        mesh=_mesh,
        scratch_types=scratch,
    )
    def sc_layer(g_hbm, r_hbm, c_hbm, norm_hbm, out_hbm, *refs):
        g_cols = refs[:cpg]
        o_cols = refs[cpg:2 * cpg]
        r0, r1, c0, c1 = refs[2 * cpg:2 * cpg + 4]
        n0, n1 = refs[2 * cpg + 4:2 * cpg + 6]
        sem0, sem1, gsem = refs[2 * cpg + 6:2 * cpg + 9]
        rbuf, cbuf, nbuf, sems = (r0, r1), (c0, c1), (n0, n1), (sem0, sem1)
        wid = _wid()
        sl_id = wid % _I(N_SLICES)
        gr = wid // _I(N_SLICES)
        ebase = sl_id * _I(SLICE_E)

        def chunk_refs(ch, b):
            cb = ebase + ch * _I(CHUNK)
            return ((r_hbm.at[pl.ds(cb, CHUNK)], rbuf[b]),
                    (c_hbm.at[pl.ds(cb, CHUNK)], cbuf[b]),
                    (norm_hbm.at[pl.ds(cb, CHUNK)], nbuf[b]))

        def start(ch, b):
            for s, d in chunk_refs(ch, b):
                pltpu.async_copy(s, d, sems[b])

        def wait(ch, b):
            for s, d in chunk_refs(ch, b):
                pltpu.make_async_copy(s, d, sems[b]).wait()

        gdescs = []
        for k in range(cpg):
            gdescs.append(pltpu.async_copy(
                g_hbm.at[pl.ds((gr * _I(cpg) + _I(k)) * _I(N), N)],
                g_cols[k], gsem))
        start(_I(0), 0)

        def zbody(i, carry):
            z = jnp.zeros((L,), jnp.float32)
            for k in range(cpg):
                o_cols[k][pl.ds(i * _I(L), L)] = z
            return carry

        lax.fori_loop(_I(0), _I(N // L), zbody, _I(0))
        for d in gdescs:
            d.wait()

        def process(b):
            def ibody(i, icarry):
                base = i * _I(UNROLL * L)
                idx = [(rbuf[b][pl.ds(base + _I(j * L), L)],
                        cbuf[b][pl.ds(base + _I(j * L), L)],
                        nbuf[b][pl.ds(base + _I(j * L), L)])
                       for j in range(UNROLL)]
                for k in range(cpg):
                    vals = [plsc.load_gather(g_cols[k], [r16]) * n16
                            for r16, _, n16 in idx]
                    for (_, c16, _), v in zip(idx, vals):
                        plsc.addupdate_scatter(o_cols[k], [c16], v)
                return icarry

            lax.fori_loop(_I(0), _I(CHUNK // (UNROLL * L)), ibody, _I(0))

        def outer(t, carry):
            for b in (0, 1):
                ch = t * _I(2) + _I(b)
                nxt = ch + _I(1)

                @pl.when(nxt < _I(N_CHUNKS))
                def _():
                    start(nxt, 1 - b)

                wait(ch, b)
                process(b)
            return carry

        lax.fori_loop(_I(0), _I(N_CHUNKS // 2), outer, _I(0))
        for k in range(cpg):
            off = (sl_id * _I(ncp) + gr * _I(cpg) + _I(k)) * _I(N)
            pltpu.sync_copy(o_cols[k], out_hbm.at[pl.ds(off, N)])

    return sc_layer


_sc_layer24 = _make_sc_layer(24)
_sc_layer16 = _make_sc_layer(16)


def _tc_prep(degp, xT, w1tp):
    def body(degp_ref, xT_ref, w_ref, g_ref, dis_ref):
        deg = jnp.sum(degp_ref[...], axis=0, keepdims=True) + 1.0
        dis_ref[...] = lax.rsqrt(deg)
        g_ref[...] = jnp.dot(w_ref[...], xT_ref[...],
                             preferred_element_type=jnp.float32, precision=lax.Precision.HIGHEST)

    return pl.pallas_call(
        body,
        out_shape=[
            jax.ShapeDtypeStruct((24, N), jnp.float32),
            jax.ShapeDtypeStruct((1, N), jnp.float32),
        ],
    )(degp, xT, w1tp)


def _tc_mid(P, gT, dis, wnt, bcol, ncp_in, ncp_out, ucol=None, wlt=None,
            blcol=None):
    def body(*refs):
        if ucol is None:
            P_ref, g_ref, dis_ref, w_ref, b_ref, o_ref = refs
        else:
            P_ref, g_ref, dis_ref, w_ref, b_ref, u_ref, wl_ref, bl_ref, o_ref = refs
        dis_v = dis_ref[...]
        s = jnp.sum(P_ref[...], axis=0) + dis_v * dis_v * g_ref[...] + b_ref[...]
        h = jnp.maximum(s, 0.0)
        if ucol is not None:
            ut = jnp.dot(wl_ref[...], u_ref[...],
                         preferred_element_type=jnp.float32, precision=lax.Precision.HIGHEST) + bl_ref[...]
            h = h + jnp.maximum(ut, 0.0)
        o_ref[...] = jnp.dot(w_ref[...], h, preferred_element_type=jnp.float32, precision=lax.Precision.HIGHEST)

    args = [P.reshape(N_SLICES, ncp_in, N), gT, dis, wnt, bcol]
    if ucol is not None:
        args += [ucol, wlt, blcol]
    return pl.pallas_call(
        body,
        out_shape=jax.ShapeDtypeStruct((ncp_out, N), jnp.float32),
    )(*args)


def _tc_final(P, gT, dis, bcol, maskT):
    def body(P_ref, g_ref, dis_ref, b_ref, m_ref, o_ref):
        dis_v = dis_ref[...]
        s = jnp.sum(P_ref[...], axis=0) + dis_v * dis_v * g_ref[...] + b_ref[...]
        o_ref[...] = s[:O] + (m_ref[...] - 1.0) * 1000.0

    return pl.pallas_call(
        body,
        out_shape=jax.ShapeDtypeStruct((O, N), jnp.float32),
    )(P.reshape(N_SLICES, 16, N), gT, dis, bcol, maskT)


def _padT(W, rows, cols):
    out = jnp.zeros((rows, cols), jnp.float32)
    return out.at[:W.shape[1], :W.shape[0]].set(W.T.astype(jnp.float32))


def _col(b, rows):
    out = jnp.zeros((rows, 1), jnp.float32)
    return out.at[:b.shape[0], 0].set(b.astype(jnp.float32))


def kernel(x, edge_index, edge_attr, u, action_mask,
           W1, b1, Wl, bl, W2, b2, W3, b3, W4, b4):
    f32 = jnp.float32
    i32 = jnp.int32
    pad = EP - E
    r = jnp.concatenate([edge_index[0].astype(i32), jnp.zeros((pad,), i32)])
    c = jnp.concatenate([edge_index[1].astype(i32), jnp.zeros((pad,), i32)])
    w = jnp.concatenate([edge_attr.astype(f32), jnp.zeros((pad,), f32)])

    xT = x.astype(f32).T
    w1tp = _padT(W1, 24, D)
    w2tp = _padT(W2, 24, 24)
    w3tp = _padT(W3, 24, 24)
    w4tp = _padT(W4, 16, 24)
    wltp = _padT(Wl, 24, 24)
    b1c, b2c, b3c = _col(b1, 24), _col(b2, 24), _col(b3, 24)
    b4c = _col(b4, 16)
    blc = _col(bl, 24)
    uc = _col(u, 24)
    maskT = action_mask.astype(f32).T

    degp = _sc_deg(c, w).reshape(NW, N)
    g1T, dis = _tc_prep(degp, xT, w1tp)
    norm = _sc_norm(r, c, w, dis.reshape(N))

    P1 = _sc_layer24(g1T.reshape(-1), r, c, norm)
    g2T = _tc_mid(P1, g1T, dis, w2tp, b1c, 24, 24, ucol=uc, wlt=wltp, blcol=blc)
    P2 = _sc_layer24(g2T.reshape(-1), r, c, norm)
    g3T = _tc_mid(P2, g2T, dis, w3tp, b2c, 24, 24)
    P3 = _sc_layer24(g3T.reshape(-1), r, c, norm)
    g4T = _tc_mid(P3, g3T, dis, w4tp, b3c, 24, 16)
    P4 = _sc_layer16(g4T.reshape(-1), r, c, norm)
    outT = _tc_final(P4, g4T, dis, b4c, maskT)
    return outT.T.astype(jnp.float64)


# speedup vs baseline: 543.0341x; 1.1674x over previous
import functools

import jax
import jax.numpy as jnp
from jax import lax
from jax.experimental import pallas as pl
from jax.experimental.pallas import tpu as pltpu
from jax.experimental.pallas import tpu_sc as plsc

N = 10000
E = 160000
D = 256
H = 22
O = 11

NC, NS, L = 2, 16, 16
NW = NC * NS
EPT = 5008
EP = EPT * NW

N_SLICES = 4
N_GROUPS = 8
SLICE_E = E // N_SLICES
CHUNK = 4000
UNROLL = 10
N_CHUNKS = SLICE_E // CHUNK

_mesh = plsc.VectorSubcoreMesh(
    core_axis_name="c", subcore_axis_name="s", num_cores=NC, num_subcores=NS)
_sc_params = pltpu.CompilerParams(needs_layout_passes=False)


_I = jnp.int32


def _wid():
    return lax.axis_index("s") * _I(NC) + lax.axis_index("c")


EPC = E // NS
N2 = 10240
STRIPE = N2 // NS

@functools.partial(
    pl.kernel,
    out_type=[
        jax.ShapeDtypeStruct((EP,), jnp.float32),
        jax.ShapeDtypeStruct((N,), jnp.float32),
    ],
    compiler_params=_sc_params,
    mesh=_mesh,
    scratch_types=[
        pltpu.VMEM_SHARED((NS * N2,), jnp.float32),
        pltpu.VMEM_SHARED((N2,), jnp.float32),
        pltpu.VMEM((N2,), jnp.float32),
        pltpu.VMEM((N2,), jnp.float32),
        pltpu.VMEM((NS * STRIPE,), jnp.float32),
        pltpu.VMEM((STRIPE,), jnp.float32),
        pltpu.VMEM((EPC,), jnp.int32),
        pltpu.VMEM((EPC,), jnp.float32),
        pltpu.VMEM((EPT,), jnp.int32),
        pltpu.VMEM((EPT,), jnp.int32),
        pltpu.VMEM((EPT,), jnp.float32),
        pltpu.VMEM((EPT,), jnp.float32),
        pltpu.SemaphoreType.DMA,
        pltpu.SemaphoreType.DMA,
    ],
)
def _sc_prep(r_hbm, c_hbm, w_hbm, norm_hbm, dis_hbm, shared_deg, shared_dis,
             deg_v, dis_v, sb_v, st_v, cd_v, wd_v, r_v, c_v, w_v, n_v,
             sem, sem2):
    sid = lax.axis_index("s")
    wid = _wid()

    db = sid * _I(EPC)
    d1 = pltpu.async_copy(c_hbm.at[pl.ds(db, EPC)], cd_v, sem)
    d2 = pltpu.async_copy(w_hbm.at[pl.ds(db, EPC)], wd_v, sem)

    def zbody(i, carry):
        deg_v[pl.ds(i * _I(L), L)] = jnp.zeros((L,), jnp.float32)
        return carry

    lax.fori_loop(_I(0), _I(N2 // L), zbody, _I(0))
    d1.wait()
    d2.wait()

    def dbody(i, carry):
        s = pl.ds(i * _I(L), L)
        plsc.addupdate_scatter(deg_v, [cd_v[s]], wd_v[s])
        return carry

    lax.fori_loop(_I(0), _I(EPC // L), dbody, _I(0))
    pltpu.sync_copy(deg_v, shared_deg.at[pl.ds(sid * _I(N2), N2)])
    plsc.subcore_barrier()

    nb = wid * _I(EPT)
    d3 = pltpu.async_copy(r_hbm.at[pl.ds(nb, EPT)], r_v, sem)
    d4 = pltpu.async_copy(c_hbm.at[pl.ds(nb, EPT)], c_v, sem)
    d5 = pltpu.async_copy(w_hbm.at[pl.ds(nb, EPT)], w_v, sem)
    sdescs = []
    for k in range(NS):
        sdescs.append(pltpu.async_copy(
            shared_deg.at[pl.ds(_I(k * N2) + sid * _I(STRIPE), STRIPE)],
            sb_v.at[pl.ds(k * STRIPE, STRIPE)], sem2))
    for d in sdescs:
        d.wait()

    def sbody(i, carry):
        s = pl.ds(i * _I(L), L)
        acc = sb_v[s]
        for k in range(1, NS):
            acc = acc + sb_v[pl.ds(_I(k * STRIPE) + i * _I(L), L)]
        d = acc + jnp.float32(1.0)
        bits = plsc.bitcast(d, jnp.int32)
        y = plsc.bitcast(_I(0x5F3759DF) - jnp.right_shift(bits, _I(1)),
                         jnp.float32)
        for _ in range(3):
            y = y * (jnp.float32(1.5) - jnp.float32(0.5) * d * y * y)
        st_v[s] = y
        return carry

    lax.fori_loop(_I(0), _I(STRIPE // L), sbody, _I(0))
    pltpu.sync_copy(st_v, shared_dis.at[pl.ds(sid * _I(STRIPE), STRIPE)])
    plsc.subcore_barrier()
    pltpu.sync_copy(shared_dis, dis_v)

    def nbody(i, carry):
        s = pl.ds(i * _I(L), L)
        dr = plsc.load_gather(dis_v, [r_v[s]])
        dc = plsc.load_gather(dis_v, [c_v[s]])
        n_v[s] = dr * w_v[s] * dc
        return carry

    d3.wait()
    d4.wait()
    d5.wait()
    lax.fori_loop(_I(0), _I(EPT // L), nbody, _I(0))
    pltpu.sync_copy(n_v, norm_hbm.at[pl.ds(nb, EPT)])

    @pl.when(wid == _I(0))
    def _():
        pltpu.sync_copy(dis_v.at[pl.ds(0, N)], dis_hbm)


def _make_sc_layer(ncp):
    cpg = ncp // N_GROUPS
    scratch = ([pltpu.VMEM((N,), jnp.float32)] * (2 * cpg)) + (
        [pltpu.VMEM((CHUNK,), jnp.int32)] * 4) + (
        [pltpu.VMEM((CHUNK,), jnp.float32)] * 2) + [
        pltpu.SemaphoreType.DMA,
        pltpu.SemaphoreType.DMA,
        pltpu.SemaphoreType.DMA,
    ]

    @functools.partial(
        pl.kernel,
        out_type=jax.ShapeDtypeStruct((N_SLICES * ncp * N,), jnp.float32),
        compiler_params=_sc_params,
        mesh=_mesh,
        scratch_types=scratch,
    )
    def sc_layer(g_hbm, r_hbm, c_hbm, norm_hbm, out_hbm, *refs):
        g_cols = refs[:cpg]
        o_cols = refs[cpg:2 * cpg]
        r0, r1, c0, c1 = refs[2 * cpg:2 * cpg + 4]
        n0, n1 = refs[2 * cpg + 4:2 * cpg + 6]
        sem0, sem1, gsem = refs[2 * cpg + 6:2 * cpg + 9]
        rbuf, cbuf, nbuf, sems = (r0, r1), (c0, c1), (n0, n1), (sem0, sem1)
        wid = _wid()
        sl_id = wid % _I(N_SLICES)
        gr = wid // _I(N_SLICES)
        ebase = sl_id * _I(SLICE_E)

        def chunk_refs(ch, b):
            cb = ebase + ch * _I(CHUNK)
            return ((r_hbm.at[pl.ds(cb, CHUNK)], rbuf[b]),
                    (c_hbm.at[pl.ds(cb, CHUNK)], cbuf[b]),
                    (norm_hbm.at[pl.ds(cb, CHUNK)], nbuf[b]))

        def start(ch, b):
            for s, d in chunk_refs(ch, b):
                pltpu.async_copy(s, d, sems[b])

        def wait(ch, b):
            for s, d in chunk_refs(ch, b):
                pltpu.make_async_copy(s, d, sems[b]).wait()

        gdescs = []
        for k in range(cpg):
            gdescs.append(pltpu.async_copy(
                g_hbm.at[pl.ds((gr * _I(cpg) + _I(k)) * _I(N), N)],
                g_cols[k], gsem))
        start(_I(0), 0)

        def zbody(i, carry):
            z = jnp.zeros((L,), jnp.float32)
            for k in range(cpg):
                o_cols[k][pl.ds(i * _I(L), L)] = z
            return carry

        lax.fori_loop(_I(0), _I(N // L), zbody, _I(0))
        for d in gdescs:
            d.wait()

        def process(b):
            def ibody(i, icarry):
                base = i * _I(UNROLL * L)
                idx = [(rbuf[b][pl.ds(base + _I(j * L), L)],
                        cbuf[b][pl.ds(base + _I(j * L), L)],
                        nbuf[b][pl.ds(base + _I(j * L), L)])
                       for j in range(UNROLL)]
                for k in range(cpg):
                    vals = [plsc.load_gather(g_cols[k], [r16]) * n16
                            for r16, _, n16 in idx]
                    for (_, c16, _), v in zip(idx, vals):
                        plsc.addupdate_scatter(o_cols[k], [c16], v)
                return icarry

            lax.fori_loop(_I(0), _I(CHUNK // (UNROLL * L)), ibody, _I(0))

        def outer(t, carry):
            for b in (0, 1):
                ch = t * _I(2) + _I(b)
                nxt = ch + _I(1)

                @pl.when(nxt < _I(N_CHUNKS))
                def _():
                    start(nxt, 1 - b)

                wait(ch, b)
                process(b)
            return carry

        lax.fori_loop(_I(0), _I(N_CHUNKS // 2), outer, _I(0))
        for k in range(cpg):
            off = (sl_id * _I(ncp) + gr * _I(cpg) + _I(k)) * _I(N)
            pltpu.sync_copy(o_cols[k], out_hbm.at[pl.ds(off, N)])

    return sc_layer


_sc_layer24 = _make_sc_layer(24)
_sc_layer16 = _make_sc_layer(16)


def _tc_g1(x, w1p):
    def body(x_ref, w_ref, g_ref):
        g = jnp.dot(x_ref[...], w_ref[...],
                    preferred_element_type=jnp.float32,
                    precision=lax.Precision.HIGHEST)
        g_ref[...] = g.T

    return pl.pallas_call(
        body,
        out_shape=jax.ShapeDtypeStruct((24, N), jnp.float32),
    )(x, w1p)


def _tc_mid(P, gT, dis, wnt, bcol, ncp_in, ncp_out, ucol=None, wlt=None,
            blcol=None):
    def body(*refs):
        if ucol is None:
            P_ref, g_ref, dis_ref, w_ref, b_ref, o_ref = refs
        else:
            P_ref, g_ref, dis_ref, w_ref, b_ref, u_ref, wl_ref, bl_ref, o_ref = refs
        dis_v = dis_ref[...]
        s = jnp.sum(P_ref[...], axis=0) + dis_v * dis_v * g_ref[...] + b_ref[...]
        h = jnp.maximum(s, 0.0)
        if ucol is not None:
            ut = jnp.dot(wl_ref[...], u_ref[...],
                         preferred_element_type=jnp.float32, precision=lax.Precision.HIGHEST) + bl_ref[...]
            h = h + jnp.maximum(ut, 0.0)
        o_ref[...] = jnp.dot(w_ref[...], h, preferred_element_type=jnp.float32, precision=lax.Precision.HIGHEST)

    args = [P.reshape(N_SLICES, ncp_in, N), gT, dis, wnt, bcol]
    if ucol is not None:
        args += [ucol, wlt, blcol]
    return pl.pallas_call(
        body,
        out_shape=jax.ShapeDtypeStruct((ncp_out, N), jnp.float32),
    )(*args)


def _tc_final(P, gT, dis, bcol, maskT):
    def body(P_ref, g_ref, dis_ref, b_ref, m_ref, o_ref):
        dis_v = dis_ref[...]
        s = jnp.sum(P_ref[...], axis=0) + dis_v * dis_v * g_ref[...] + b_ref[...]
        o_ref[...] = s[:O] + (m_ref[...] - 1.0) * 1000.0

    return pl.pallas_call(
        body,
        out_shape=jax.ShapeDtypeStruct((O, N), jnp.float32),
    )(P.reshape(N_SLICES, 16, N), gT, dis, bcol, maskT)


def _padT(W, rows, cols):
    out = jnp.zeros((rows, cols), jnp.float32)
    return out.at[:W.shape[1], :W.shape[0]].set(W.T.astype(jnp.float32))


def _col(b, rows):
    out = jnp.zeros((rows, 1), jnp.float32)
    return out.at[:b.shape[0], 0].set(b.astype(jnp.float32))


def kernel(x, edge_index, edge_attr, u, action_mask,
           W1, b1, Wl, bl, W2, b2, W3, b3, W4, b4):
    f32 = jnp.float32
    i32 = jnp.int32
    pad = EP - E
    r = jnp.concatenate([edge_index[0].astype(i32), jnp.zeros((pad,), i32)])
    c = jnp.concatenate([edge_index[1].astype(i32), jnp.zeros((pad,), i32)])
    w = jnp.concatenate([edge_attr.astype(f32), jnp.zeros((pad,), f32)])

    xp = x.astype(f32)
    w1p = jnp.zeros((D, 24), f32).at[:, :H].set(W1.astype(f32))
    w2tp = _padT(W2, 24, 24)
    w3tp = _padT(W3, 24, 24)
    w4tp = _padT(W4, 16, 24)
    wltp = _padT(Wl, 24, 24)
    b1c, b2c, b3c = _col(b1, 24), _col(b2, 24), _col(b3, 24)
    b4c = _col(b4, 16)
    blc = _col(bl, 24)
    uc = _col(u, 24)
    maskT = action_mask.astype(f32).T

    norm, dis1 = _sc_prep(r, c, w)
    g1T = _tc_g1(xp, w1p)
    dis = dis1.reshape(1, N)

    P1 = _sc_layer24(g1T.reshape(-1), r, c, norm)
    g2T = _tc_mid(P1, g1T, dis, w2tp, b1c, 24, 24, ucol=uc, wlt=wltp, blcol=blc)
    P2 = _sc_layer24(g2T.reshape(-1), r, c, norm)
    g3T = _tc_mid(P2, g2T, dis, w3tp, b2c, 24, 24)
    P3 = _sc_layer24(g3T.reshape(-1), r, c, norm)
    g4T = _tc_mid(P3, g3T, dis, w4tp, b3c, 24, 16)
    P4 = _sc_layer16(g4T.reshape(-1), r, c, norm)
    outT = _tc_final(P4, g4T, dis, b4c, maskT)
    return outT.T.astype(jnp.float64)
